# baseline (speedup 1.0000x reference)
import jax
import jax.numpy as jnp
from jax.experimental import pallas as pl
from jax.experimental.pallas import tpu as pltpu

_RADIUS = 0.42
_BRIDGE_SCALE = 0.18
_BRIDGE_TOPK = 6
_NEG = -1000000000.0


_RHS_CONTRACT = (((1,), (1,)), ((), ()))


def _fused(tok_ref, prev_ref, consts_ref, ws_ref, wb_ref, posm_ref, pv_ref,
           w_ref, ps_ref, stats_ref):
    f32 = jnp.float32
    bf16 = jnp.bfloat16
    q = jnp.concatenate(
        [tok_ref[...].astype(bf16), prev_ref[...].astype(bf16)], axis=1)
    consts = consts_ref[...]
    px = consts[0:1, :]
    py = consts[1:2, :]
    pz = consts[2:3, :]
    sel_b = consts[3:4, :]
    br_b = consts[4:5, :]

    bls = jax.lax.dot_general(q, ws_ref[...], _RHS_CONTRACT,
                              preferred_element_type=f32) + sel_b
    m = jnp.max(bls, axis=1, keepdims=True)
    e = jnp.exp(bls - m)
    s = jnp.sum(e, axis=1, keepdims=True)
    bw = e * (1.0 / s)

    anch = jnp.dot(e.astype(jnp.bfloat16), posm_ref[...],
                   preferred_element_type=f32)
    ax = anch[:, 0:1] / s
    ay = anch[:, 1:2] / s
    az = anch[:, 2:3] / s
    dx = ax - px
    dy = ay - py
    dz = az - pz
    d2 = jnp.maximum(dx * dx + dy * dy + dz * dz, 1e-12)
    dist = jnp.sqrt(d2)

    lmask = dist <= _RADIUS
    lk = jnp.exp(d2 * (-1.0 / (2.0 * _RADIUS * _RADIUS)))

    blb = jax.lax.dot_general(q, wb_ref[...], _RHS_CONTRACT,
                              preferred_element_type=f32) + br_b
    blm = jnp.where(lmask, _NEG, blb)

    cur = blm
    m0 = jnp.max(cur, axis=1, keepdims=True)
    mi = m0
    for _ in range(_BRIDGE_TOPK - 1):
        cur = jnp.where(cur >= mi, -2.0e9, cur)
        mi = jnp.max(cur, axis=1, keepdims=True)

    sexp = jnp.where(blm >= mi, jnp.exp(blm - m0), 0.0)
    sparse = sexp / jnp.sum(sexp, axis=1, keepdims=True)

    lmf = lmask.astype(f32)
    mixed = bw * lk * lmf + _BRIDGE_SCALE * sparse
    z = jnp.maximum(jnp.sum(mixed, axis=1, keepdims=True), 1e-6)
    w = mixed / z

    w_ref[...] = w
    ps_ref[...] = jnp.dot(w.astype(jnp.bfloat16), pv_ref[...],
                          preferred_element_type=f32)
    stats_ref[:, 0:1] = ax
    stats_ref[:, 1:2] = ay
    stats_ref[:, 2:3] = az
    stats_ref[:, 3:4] = jnp.sum(w * lmf, axis=1, keepdims=True)
    stats_ref[:, 4:5] = jnp.sum(w * (1.0 - lmf), axis=1, keepdims=True)
    stats_ref[:, 5:6] = jnp.sum(w * dist, axis=1, keepdims=True)
    stats_ref[:, 6:7] = jnp.full(ax.shape, _RADIUS, f32)
    stats_ref[:, 7:8] = jnp.full(ax.shape, _BRIDGE_SCALE, f32)


def kernel(token_state, prev_state, patch_values, sel_W, sel_b, br_W, br_b,
           positions, block_rows=256, interpret=False):
    n, h = token_state.shape
    p = patch_values.shape[0]
    f32 = jnp.float32
    bf16 = jnp.bfloat16

    ws16 = sel_W.astype(bf16)
    wb16 = br_W.astype(bf16)
    pv16 = patch_values.astype(bf16)
    consts = jnp.stack([
        positions[:, 0], positions[:, 1], positions[:, 2],
        sel_b, br_b,
        jnp.zeros((p,), f32), jnp.zeros((p,), f32), jnp.zeros((p,), f32)],
        axis=0)
    posm16 = jnp.zeros((p, 128), bf16).at[:, :3].set(positions.astype(bf16))

    R = block_rows
    nb = n // R
    out_shape = (
        jax.ShapeDtypeStruct((n, p), f32),
        jax.ShapeDtypeStruct((n, h), f32),
        jax.ShapeDtypeStruct((n, 8), f32),
    )
    row_spec = lambda c: pl.BlockSpec((R, c), lambda i: (i, 0))
    full_spec = lambda r, c: pl.BlockSpec((r, c), lambda i: (0, 0))
    weights, patch_state, stats = (
        pl.pallas_call(
            _fused,
            grid=(nb,),
            in_specs=[
                row_spec(h), row_spec(h),
                full_spec(8, p),
                full_spec(p, 2 * h), full_spec(p, 2 * h),
                full_spec(p, 128),
                full_spec(p, h),
            ],
            out_specs=(row_spec(p), row_spec(h), row_spec(8)),
            out_shape=out_shape,
            compiler_params=pltpu.CompilerParams(
                dimension_semantics=("arbitrary",)),
            interpret=interpret,
        )(token_state, prev_state, consts, ws16, wb16, posm16, pv16)
    )

    return (patch_state, weights, stats[:, 0:3], stats[:, 3:4],
            stats[:, 4:5], stats[:, 5:6], stats[:, 6:7], stats[:, 7:8])


# speedup vs baseline: 1.1966x; 1.1966x over previous
import jax
import jax.numpy as jnp
from jax.experimental import pallas as pl
from jax.experimental.pallas import tpu as pltpu

_RADIUS = 0.42
_BRIDGE_SCALE = 0.18
_BRIDGE_TOPK = 6
_NEG = -1000000000.0


_RHS_CONTRACT = (((1,), (1,)), ((), ()))


def _fused(tok_ref, prev_ref, consts_ref, ws_ref, wb_ref, posm_ref, pv_ref,
           w_ref, ps_ref, anchor_ref, lm_ref, bm_ref, md_ref):
    f32 = jnp.float32
    bf16 = jnp.bfloat16
    q = jnp.concatenate(
        [tok_ref[...].astype(bf16), prev_ref[...].astype(bf16)], axis=1)
    consts = consts_ref[...]
    px = consts[0:1, :]
    py = consts[1:2, :]
    pz = consts[2:3, :]
    sel_b = consts[3:4, :]
    br_b = consts[4:5, :]

    bls = jax.lax.dot_general(q, ws_ref[...], _RHS_CONTRACT,
                              preferred_element_type=f32) + sel_b
    m = jnp.max(bls, axis=1, keepdims=True)
    e = jnp.exp(bls - m)
    s = jnp.sum(e, axis=1, keepdims=True)
    bw = e * (1.0 / s)

    anch = jnp.dot(e.astype(jnp.bfloat16), posm_ref[...],
                   preferred_element_type=f32)
    ax = anch[:, 0:1] / s
    ay = anch[:, 1:2] / s
    az = anch[:, 2:3] / s
    dx = ax - px
    dy = ay - py
    dz = az - pz
    d2 = jnp.maximum(dx * dx + dy * dy + dz * dz, 1e-12)
    dist = jnp.sqrt(d2)

    lmask = dist <= _RADIUS
    lk = jnp.exp(d2 * (-1.0 / (2.0 * _RADIUS * _RADIUS)))

    blb = jax.lax.dot_general(q, wb_ref[...], _RHS_CONTRACT,
                              preferred_element_type=f32) + br_b
    blm = jnp.where(lmask, _NEG, blb)

    cur = blm
    m0 = jnp.max(cur, axis=1, keepdims=True)
    mi = m0
    for _ in range(_BRIDGE_TOPK - 1):
        cur = jnp.where(cur >= mi, -2.0e9, cur)
        mi = jnp.max(cur, axis=1, keepdims=True)

    sexp = jnp.where(blm >= mi, jnp.exp(blm - m0), 0.0)
    sparse = sexp / jnp.sum(sexp, axis=1, keepdims=True)

    lmf = lmask.astype(f32)
    mixed = bw * lk * lmf + _BRIDGE_SCALE * sparse
    z = jnp.maximum(jnp.sum(mixed, axis=1, keepdims=True), 1e-6)
    w = mixed / z

    w_ref[...] = w
    ps_ref[...] = jnp.dot(w.astype(jnp.bfloat16), pv_ref[...],
                          preferred_element_type=f32)
    anchor_ref[:, 0:1] = ax
    anchor_ref[:, 1:2] = ay
    anchor_ref[:, 2:3] = az
    lm_ref[...] = jnp.sum(w * lmf, axis=1, keepdims=True)
    bm_ref[...] = jnp.sum(w * (1.0 - lmf), axis=1, keepdims=True)
    md_ref[...] = jnp.sum(w * dist, axis=1, keepdims=True)


def kernel(token_state, prev_state, patch_values, sel_W, sel_b, br_W, br_b,
           positions, block_rows=256, interpret=False):
    n, h = token_state.shape
    p = patch_values.shape[0]
    f32 = jnp.float32
    bf16 = jnp.bfloat16

    ws16 = sel_W.astype(bf16)
    wb16 = br_W.astype(bf16)
    pv16 = patch_values.astype(bf16)
    consts = jnp.stack([
        positions[:, 0], positions[:, 1], positions[:, 2],
        sel_b, br_b,
        jnp.zeros((p,), f32), jnp.zeros((p,), f32), jnp.zeros((p,), f32)],
        axis=0)
    posm16 = jnp.zeros((p, 128), bf16).at[:, :3].set(positions.astype(bf16))

    R = block_rows
    nb = n // R
    out_shape = (
        jax.ShapeDtypeStruct((n, p), f32),
        jax.ShapeDtypeStruct((n, h), f32),
        jax.ShapeDtypeStruct((n, 3), f32),
        jax.ShapeDtypeStruct((n, 1), f32),
        jax.ShapeDtypeStruct((n, 1), f32),
        jax.ShapeDtypeStruct((n, 1), f32),
    )
    row_spec = lambda c: pl.BlockSpec((R, c), lambda i: (i, 0))
    full_spec = lambda r, c: pl.BlockSpec((r, c), lambda i: (0, 0))
    weights, patch_state, anchor_pos, local_mass, bridge_mass, mean_distance = (
        pl.pallas_call(
            _fused,
            grid=(nb,),
            in_specs=[
                row_spec(h), row_spec(h),
                full_spec(8, p),
                full_spec(p, 2 * h), full_spec(p, 2 * h),
                full_spec(p, 128),
                full_spec(p, h),
            ],
            out_specs=(
                row_spec(p), row_spec(h), row_spec(3),
                row_spec(1), row_spec(1), row_spec(1),
            ),
            out_shape=out_shape,
            compiler_params=pltpu.CompilerParams(
                dimension_semantics=("arbitrary",)),
            interpret=interpret,
        )(token_state, prev_state, consts, ws16, wb16, posm16, pv16)
    )

    radius = jnp.full((n, 1), _RADIUS, dtype=f32)
    bridge_scale = jnp.full((n, 1), _BRIDGE_SCALE, dtype=f32)
    return (patch_state, weights, anchor_pos, local_mass, bridge_mass,
            mean_distance, radius, bridge_scale)
